# Initial kernel scaffold; baseline (speedup 1.0000x reference)
import jax
import jax.numpy as jnp
from jax.experimental import pallas as pl


def kernel(x, w0, b0, w1, b1, w2, b2, w3, b3, w4, b4, w5, b5, w6, b6, w7, b7, w8, b8):
    raise NotImplementedError("write your pallas kernel here")


# speedup vs baseline: 2.7415x; 2.7415x over previous
import numpy as np
import jax
import jax.numpy as jnp
from jax import lax
from jax.experimental import pallas as pl
from jax.experimental.pallas import tpu as pltpu

_B = 8


def _interp_mat(n_in, n_out):
    pos = np.arange(n_out) * (n_in - 1) / (n_out - 1)
    lo = np.floor(pos).astype(np.int64)
    hi = np.minimum(lo + 1, n_in - 1)
    f = (pos - lo).astype(np.float32)
    a = np.zeros((n_in, n_out), np.float32)
    a[lo, np.arange(n_out)] += 1.0 - f
    a[hi, np.arange(n_out)] += f
    return a


def _kron_mat(h, w):
    ah = _interp_mat(h, 2 * h).T
    aw = _interp_mat(w, 2 * w).T
    return jnp.asarray(np.kron(ah, aw).T)


def _contract(lhs, rhs, lhs_dim, rhs_dim=0):
    return lax.dot_general(lhs, rhs, (((lhs_dim,), (rhs_dim,)), ((), ())),
                           preferred_element_type=jnp.float32)


def _fused_body(x_ref,
                w0, b0, w1, b1, w2, b2, w3, b3, w4, b4, w5, b5,
                w6, b6, w7, b7, w8, b8,
                kt1, kt2, a3, a4, o_ref):
    B = x_ref.shape[0]

    def conv(w, b, y, relu=True):
        y = _contract(w[...], y, 1, 0) + b[...][:, :, None]
        return jnp.maximum(y, 0.0) if relu else y

    y = jnp.transpose(x_ref[...], (1, 0, 2))
    y = conv(w0, b0, y)
    y = conv(w1, b1, y)
    y = _contract(y, kt1[...], 2)
    y = conv(w2, b2, y)
    y = conv(w3, b3, y)
    y = _contract(y, kt2[...], 2)
    y = conv(w4, b4, y)
    y = conv(w5, b5, y)
    y = y.reshape(32, B, 32, 32)
    y = _contract(y, a3[...], 3)
    y = _contract(y, a3[...], 2)
    y = y.reshape(32, B, 4096)
    y = conv(w6, b6, y)
    y = conv(w7, b7, y)
    y = conv(w8, b8, y, relu=False)
    y = y.reshape(2, B, 64, 64)
    y = _contract(y, a4[...], 3)
    y = _contract(y, a4[...], 2)
    o_ref[:, 0] = y[0]
    o_ref[:, 1] = y[1]


def kernel(x, w0, b0, w1, b1, w2, b2, w3, b3, w4, b4, w5, b5, w6, b6,
           w7, b7, w8, b8):
    n = x.shape[0]
    B = _B
    xc = x.reshape(n, 128, 64).astype(jnp.float32)

    consts = [_kron_mat(8, 8), _kron_mat(16, 16),
              jnp.asarray(_interp_mat(32, 64)), jnp.asarray(_interp_mat(64, 128))]
    params = [w0, b0, w1, b1, w2, b2, w3, b3, w4, b4, w5, b5,
              w6, b6, w7, b7, w8, b8]

    in_specs = [pl.BlockSpec((B, 128, 64), lambda i: (i, 0, 0))]
    for a in params + consts:
        in_specs.append(pl.BlockSpec(a.shape, (lambda i: (0, 0))))

    out = pl.pallas_call(
        _fused_body,
        out_shape=jax.ShapeDtypeStruct((n, 2, 128, 128), jnp.float32),
        grid=(n // B,),
        in_specs=in_specs,
        out_specs=pl.BlockSpec((B, 2, 128, 128), lambda i: (i, 0, 0, 0)),
        compiler_params=pltpu.CompilerParams(
            dimension_semantics=("parallel",)),
    )(xc, *params, *consts)
    return out


# speedup vs baseline: 5.2413x; 1.9119x over previous
import numpy as np
import jax
import jax.numpy as jnp
from jax import lax
from jax.experimental import pallas as pl
from jax.experimental.pallas import tpu as pltpu

_B = 8


def _interp_mat(n_in, n_out):
    pos = np.arange(n_out) * (n_in - 1) / (n_out - 1)
    lo = np.floor(pos).astype(np.int64)
    hi = np.minimum(lo + 1, n_in - 1)
    f = (pos - lo).astype(np.float32)
    a = np.zeros((n_in, n_out), np.float32)
    a[lo, np.arange(n_out)] += 1.0 - f
    a[hi, np.arange(n_out)] += f
    return a


def _kron_mat(h, w):
    ah = _interp_mat(h, 2 * h).T
    aw = _interp_mat(w, 2 * w).T
    return jnp.asarray(np.kron(ah, aw).T)


def _dot2d(w, y):
    return jnp.dot(w, y, preferred_element_type=jnp.float32)


def _dot_minor(y, a):
    return lax.dot_general(y, a, (((y.ndim - 1,), (0,)), ((), ())),
                           preferred_element_type=jnp.float32)


def _up_kron(y, kt, b, m):
    outs = [_dot2d(y[:, s * m:(s + 1) * m], kt) for s in range(b)]
    return jnp.concatenate(outs, axis=1)


def _h_weights(shape, w, n, flip=False):
    lane = lax.broadcasted_iota(jnp.int32, shape, 1)
    k = ((lane // w) % n).astype(jnp.float32)
    if flip:
        k = (n - 1.0) - k
    return k / (2.0 * n - 1.0)


def _shl(y, w):
    return jnp.concatenate([y[:, w:], y[:, -w:]], axis=1)


def _shr(y, w):
    return jnp.concatenate([y[:, :w], y[:, :-w]], axis=1)


def _fused_body(x_ref,
                w0, b0, w1, b1, w2, b2, w3, b3, w4, b4, w5, b5,
                w6, b6, w7, b7, w8, b8,
                kt1, kt2, a3, a4, o_ref):
    B = x_ref.shape[0]

    def conv(w, b, y, relu=True):
        y = _dot2d(w[...], y) + b[...]
        return jnp.maximum(y, 0.0) if relu else y

    y = jnp.transpose(x_ref[...], (1, 0, 2)).reshape(128, B * 64)
    y = conv(w0, b0, y)
    y = conv(w1, b1, y)
    y = _up_kron(y, kt1[...], B, 64)
    y = conv(w2, b2, y)
    y = conv(w3, b3, y)
    y = _up_kron(y, kt2[...], B, 256)
    y = conv(w4, b4, y)
    y = conv(w5, b5, y)

    y = _dot_minor(y.reshape(32, B, 32, 32), a3[...])
    y = y.reshape(32, B * 2048)
    a = _h_weights(y.shape, 64, 32)
    g = _h_weights(y.shape, 64, 32, flip=True)
    ye = y * (1.0 - a) + _shr(y, 64) * a
    yo = y * (1.0 - g) + _shl(y, 64) * g
    y = jnp.concatenate([ye, yo], axis=1)

    y = conv(w6, b6, y)
    y = conv(w7, b7, y)
    y = conv(w8, b8, y, relu=False)

    half = B * 2048
    E, O = y[:, :half], y[:, half:]
    lane = lax.broadcasted_iota(jnp.int32, E.shape, 1)
    k2 = 2.0 * ((lane // 64) % 32).astype(jnp.float32)
    s = 1.0 / 127.0
    a0 = k2 * s
    g0 = (63.0 - k2) * s
    a1 = (k2 + 1.0) * s
    g1 = (62.0 - k2) * s
    p0 = _shr(O, 64) * a0 + E * (1.0 - a0)
    p1 = E * (1.0 - g0) + O * g0
    p2 = E * a1 + O * (1.0 - a1)
    p3 = O * (1.0 - g1) + _shl(E, 64) * g1
    y = jnp.concatenate([p0, p1, p2, p3], axis=1)
    y = y.reshape(2, 4, B, 32, 64)
    y = _dot_minor(y, a4[...])

    for c in range(2):
        for p in range(4):
            o_ref[:, c, :, p, :] = y[c, p]


def kernel(x, w0, b0, w1, b1, w2, b2, w3, b3, w4, b4, w5, b5, w6, b6,
           w7, b7, w8, b8):
    n = x.shape[0]
    B = _B
    xc = x.reshape(n, 128, 64).astype(jnp.float32)

    consts = [_kron_mat(8, 8), _kron_mat(16, 16),
              jnp.asarray(_interp_mat(32, 64)), jnp.asarray(_interp_mat(64, 128))]
    params = [w0, b0, w1, b1, w2, b2, w3, b3, w4, b4, w5, b5,
              w6, b6, w7, b7, w8, b8]

    in_specs = [pl.BlockSpec((B, 128, 64), lambda i: (i, 0, 0))]
    for a in params + consts:
        in_specs.append(pl.BlockSpec(a.shape, (lambda i: (0, 0))))

    out = pl.pallas_call(
        _fused_body,
        out_shape=jax.ShapeDtypeStruct((n, 2, 32, 4, 128), jnp.float32),
        grid=(n // B,),
        in_specs=in_specs,
        out_specs=pl.BlockSpec((B, 2, 32, 4, 128), lambda i: (i, 0, 0, 0, 0)),
        compiler_params=pltpu.CompilerParams(
            dimension_semantics=("parallel",)),
    )(xc, *params, *consts)
    return out.reshape(n, 2, 128, 128)


# speedup vs baseline: 5.4354x; 1.0370x over previous
import numpy as np
import jax
import jax.numpy as jnp
from jax import lax
from jax.experimental import pallas as pl
from jax.experimental.pallas import tpu as pltpu

_B = 8


def _interp_mat(n_in, n_out):
    pos = np.arange(n_out) * (n_in - 1) / (n_out - 1)
    lo = np.floor(pos).astype(np.int64)
    hi = np.minimum(lo + 1, n_in - 1)
    f = (pos - lo).astype(np.float32)
    a = np.zeros((n_in, n_out), np.float32)
    a[lo, np.arange(n_out)] += 1.0 - f
    a[hi, np.arange(n_out)] += f
    return a


def _kron_mat(h, w):
    ah = _interp_mat(h, 2 * h).T
    aw = _interp_mat(w, 2 * w).T
    return jnp.asarray(np.kron(ah, aw).T)


def _dot2d(w, y):
    return jnp.dot(w, y, preferred_element_type=jnp.float32)


def _dot_minor(y, a):
    return lax.dot_general(y, a, (((y.ndim - 1,), (0,)), ((), ())),
                           preferred_element_type=jnp.float32)


def _up_kron(y, kt, b, m):
    outs = [_dot2d(y[:, s * m:(s + 1) * m], kt).astype(jnp.bfloat16)
            for s in range(b)]
    return jnp.concatenate(outs, axis=1)


def _h_weights(shape, w, n, flip=False):
    lane = lax.broadcasted_iota(jnp.int32, shape, 1)
    k = ((lane // w) % n).astype(jnp.float32)
    if flip:
        k = (n - 1.0) - k
    return k / (2.0 * n - 1.0)


def _shl(y, w):
    return jnp.concatenate([y[:, w:], y[:, -w:]], axis=1)


def _shr(y, w):
    return jnp.concatenate([y[:, :w], y[:, :-w]], axis=1)


def _fused_body(x_ref,
                w0, b0, w1, b1, w2, b2, w3, b3, w4, b4, w5, b5,
                w6, b6, w7, b7, w8, b8,
                kt1, kt2, a3, a4, o_ref):
    B = x_ref.shape[0]

    def conv(w, b, y, relu=True):
        y = _dot2d(w[...], y) + b[...]
        if relu:
            y = jnp.maximum(y, 0.0)
        return y.astype(jnp.bfloat16)

    y = jnp.transpose(x_ref[...], (1, 0, 2)).reshape(128, B * 64)
    y = conv(w0, b0, y)
    y = conv(w1, b1, y)
    y = _up_kron(y, kt1[...], B, 64)
    y = conv(w2, b2, y)
    y = conv(w3, b3, y)
    y = _up_kron(y, kt2[...], B, 256)
    y = conv(w4, b4, y)
    y = conv(w5, b5, y)

    y = _dot_minor(y.reshape(32, B, 32, 32), a3[...])
    y = y.astype(jnp.bfloat16).reshape(32, B * 2048)
    a = _h_weights(y.shape, 64, 32).astype(jnp.bfloat16)
    g = _h_weights(y.shape, 64, 32, flip=True).astype(jnp.bfloat16)
    one = jnp.bfloat16(1.0)
    ye = y * (one - a) + _shr(y, 64) * a
    yo = y * (one - g) + _shl(y, 64) * g
    y = jnp.concatenate([ye, yo], axis=1)

    y = conv(w6, b6, y)
    y = conv(w7, b7, y)
    y = conv(w8, b8, y, relu=False)

    half = B * 2048
    E, O = y[:, :half], y[:, half:]
    lane = lax.broadcasted_iota(jnp.int32, E.shape, 1)
    k2 = 2.0 * ((lane // 64) % 32).astype(jnp.float32)
    s = 1.0 / 127.0
    a0 = (k2 * s).astype(jnp.bfloat16)
    g0 = ((63.0 - k2) * s).astype(jnp.bfloat16)
    a1 = ((k2 + 1.0) * s).astype(jnp.bfloat16)
    g1 = ((62.0 - k2) * s).astype(jnp.bfloat16)
    one = jnp.bfloat16(1.0)
    p0 = _shr(O, 64) * a0 + E * (one - a0)
    p1 = E * (one - g0) + O * g0
    p2 = E * a1 + O * (one - a1)
    p3 = O * (one - g1) + _shl(E, 64) * g1
    y = jnp.concatenate([p0, p1, p2, p3], axis=1)
    y = y.reshape(2, 4, B, 32, 64)
    y = _dot_minor(y, a4[...])

    for c in range(2):
        for p in range(4):
            o_ref[:, c, :, p, :] = y[c, p]


def kernel(x, w0, b0, w1, b1, w2, b2, w3, b3, w4, b4, w5, b5, w6, b6,
           w7, b7, w8, b8):
    n = x.shape[0]
    B = _B
    xc = x.reshape(n, 128, 64).astype(jnp.bfloat16)

    consts = [_kron_mat(8, 8), _kron_mat(16, 16),
              jnp.asarray(_interp_mat(32, 64)), jnp.asarray(_interp_mat(64, 128))]
    consts = [c.astype(jnp.bfloat16) for c in consts]
    params = [w0, b0, w1, b1, w2, b2, w3, b3, w4, b4, w5, b5,
              w6, b6, w7, b7, w8, b8]
    params = [p.astype(jnp.bfloat16) if p.shape[-1] != 1 else p
              for p in params]

    in_specs = [pl.BlockSpec((B, 128, 64), lambda i: (i, 0, 0))]
    for a in params + consts:
        in_specs.append(pl.BlockSpec(a.shape, (lambda i: (0, 0))))

    out = pl.pallas_call(
        _fused_body,
        out_shape=jax.ShapeDtypeStruct((n, 2, 32, 4, 128), jnp.float32),
        grid=(n // B,),
        in_specs=in_specs,
        out_specs=pl.BlockSpec((B, 2, 32, 4, 128), lambda i: (i, 0, 0, 0, 0)),
        compiler_params=pltpu.CompilerParams(
            dimension_semantics=("parallel",)),
    )(xc, *params, *consts)
    return out.reshape(n, 2, 128, 128)


# speedup vs baseline: 5.7890x; 1.0651x over previous
import numpy as np
import jax
import jax.numpy as jnp
from jax import lax
from jax.experimental import pallas as pl
from jax.experimental.pallas import tpu as pltpu

_B = 16


def _interp_mat(n_in, n_out):
    pos = np.arange(n_out) * (n_in - 1) / (n_out - 1)
    lo = np.floor(pos).astype(np.int64)
    hi = np.minimum(lo + 1, n_in - 1)
    f = (pos - lo).astype(np.float32)
    a = np.zeros((n_in, n_out), np.float32)
    a[lo, np.arange(n_out)] += 1.0 - f
    a[hi, np.arange(n_out)] += f
    return a


def _kron_mat(h, w):
    ah = _interp_mat(h, 2 * h).T
    aw = _interp_mat(w, 2 * w).T
    return jnp.asarray(np.kron(ah, aw).T)


def _dot2d(w, y):
    return jnp.dot(w, y, preferred_element_type=jnp.float32)


def _dot_minor(y, a):
    return lax.dot_general(y, a, (((y.ndim - 1,), (0,)), ((), ())),
                           preferred_element_type=jnp.float32)


def _up_kron(y, kt, b, m):
    c = y.shape[0]
    out = _dot_minor(y.reshape(c, b, m), kt)
    return out.astype(jnp.bfloat16).reshape(c, b * 4 * m)


def _h_weights(shape, w, n, flip=False):
    lane = lax.broadcasted_iota(jnp.int32, shape, 1)
    k = ((lane // w) % n).astype(jnp.float32)
    if flip:
        k = (n - 1.0) - k
    return k / (2.0 * n - 1.0)


def _shl(y, w):
    return jnp.concatenate([y[:, w:], y[:, -w:]], axis=1)


def _shr(y, w):
    return jnp.concatenate([y[:, :w], y[:, :-w]], axis=1)


def _fused_body(x_ref,
                w0, b0, w1, b1, w2, b2, w3, b3, w4, b4, w5, b5,
                w6, b6, w7, b7, w8, b8,
                kt1, kt2, a3, a4, o_ref):
    B = x_ref.shape[0]

    def conv(w, b, y, relu=True):
        y = _dot2d(w[...], y) + b[...]
        if relu:
            y = jnp.maximum(y, 0.0)
        return y.astype(jnp.bfloat16)

    y = jnp.transpose(x_ref[...], (1, 0, 2)).reshape(128, B * 64)
    y = conv(w0, b0, y)
    y = conv(w1, b1, y)
    y = _up_kron(y, kt1[...], B, 64)
    y = conv(w2, b2, y)
    y = conv(w3, b3, y)
    y = _up_kron(y, kt2[...], B, 256)
    y = conv(w4, b4, y)
    y = conv(w5, b5, y)

    y = _dot_minor(y.reshape(32, B, 32, 32), a3[...])
    y = y.astype(jnp.bfloat16).reshape(32, B * 2048)
    a = _h_weights(y.shape, 64, 32).astype(jnp.bfloat16)
    g = _h_weights(y.shape, 64, 32, flip=True).astype(jnp.bfloat16)
    one = jnp.bfloat16(1.0)
    ye = y * (one - a) + _shr(y, 64) * a
    yo = y * (one - g) + _shl(y, 64) * g
    y = jnp.concatenate([ye, yo], axis=1)

    y = conv(w6, b6, y)
    y = conv(w7, b7, y)
    y = conv(w8, b8, y, relu=False)

    half = B * 2048
    E, O = y[:, :half], y[:, half:]
    lane = lax.broadcasted_iota(jnp.int32, E.shape, 1)
    k2 = 2.0 * ((lane // 64) % 32).astype(jnp.float32)
    s = 1.0 / 127.0
    a0 = (k2 * s).astype(jnp.bfloat16)
    g0 = ((63.0 - k2) * s).astype(jnp.bfloat16)
    a1 = ((k2 + 1.0) * s).astype(jnp.bfloat16)
    g1 = ((62.0 - k2) * s).astype(jnp.bfloat16)
    one = jnp.bfloat16(1.0)
    p0 = _shr(O, 64) * a0 + E * (one - a0)
    p1 = E * (one - g0) + O * g0
    p2 = E * a1 + O * (one - a1)
    p3 = O * (one - g1) + _shl(E, 64) * g1
    y = jnp.concatenate([p0, p1, p2, p3], axis=1)
    y = y.reshape(2, 4, B, 32, 64)
    y = _dot_minor(y, a4[...])

    for c in range(2):
        for p in range(4):
            o_ref[:, c, :, p, :] = y[c, p]


def kernel(x, w0, b0, w1, b1, w2, b2, w3, b3, w4, b4, w5, b5, w6, b6,
           w7, b7, w8, b8):
    n = x.shape[0]
    B = _B
    xc = x.reshape(n, 128, 64).astype(jnp.bfloat16)

    consts = [_kron_mat(8, 8), _kron_mat(16, 16),
              jnp.asarray(_interp_mat(32, 64)), jnp.asarray(_interp_mat(64, 128))]
    consts = [c.astype(jnp.bfloat16) for c in consts]
    params = [w0, b0, w1, b1, w2, b2, w3, b3, w4, b4, w5, b5,
              w6, b6, w7, b7, w8, b8]
    params = [p.astype(jnp.bfloat16) if p.shape[-1] != 1 else p
              for p in params]

    in_specs = [pl.BlockSpec((B, 128, 64), lambda i: (i, 0, 0))]
    for a in params + consts:
        in_specs.append(pl.BlockSpec(a.shape, (lambda i: (0, 0))))

    out = pl.pallas_call(
        _fused_body,
        out_shape=jax.ShapeDtypeStruct((n, 2, 32, 4, 128), jnp.float32),
        grid=(n // B,),
        in_specs=in_specs,
        out_specs=pl.BlockSpec((B, 2, 32, 4, 128), lambda i: (i, 0, 0, 0, 0)),
        compiler_params=pltpu.CompilerParams(
            dimension_semantics=("parallel",)),
    )(xc, *params, *consts)
    return out.reshape(n, 2, 128, 128)
